# baseline (speedup 1.0000x reference)
import jax
import jax.numpy as jnp
from jax.experimental import pallas as pl
from jax.experimental.pallas import tpu as pltpu
from jax.experimental.pallas import tpu_sc as plsc

NC, NS, LANES = 2, 16, 16
NW = NC * NS
B, L = 16384, 64
BLK = 128
N_BLKS = B // BLK
BLK_PER_W = N_BLKS // NW
NBUF = 2
TBL = 6 * 2 * 512
VECS = L * BLK // LANES


def _body(piece_hbm, color_hbm, pos_hbm, wp_hbm, wc_hbm, wq_hbm, out_hbm,
          tbl_p, pv0, cv0, qv0, ob0, pv1, cv1, qv1, ob1,
          wp_v, wc_v, wq_v, sin0, sin1, sout0, sout1):
    wid = jax.lax.axis_index("s") * NC + jax.lax.axis_index("c")
    iota = jax.lax.iota(jnp.int32, LANES)

    bufs = [(pv0, cv0, qv0, ob0, sin0, sout0),
            (pv1, cv1, qv1, ob1, sin1, sout1)]

    def start_in(k):
        pv, cv, qv, _, sin, _ = bufs[k % NBUF]
        bc = (wid * BLK_PER_W + k) * BLK
        sl = pl.ds(bc, BLK)
        return [pltpu.async_copy(piece_hbm.at[:, sl], pv, sin),
                pltpu.async_copy(color_hbm.at[:, sl], cv, sin),
                pltpu.async_copy(pos_hbm.at[:, sl], qv, sin)]

    in_descs = {0: start_in(0), 1: start_in(1)}

    pltpu.sync_copy(wp_hbm, wp_v)
    pltpu.sync_copy(wc_hbm, wc_v)
    pltpu.sync_copy(wq_hbm, wq_v)

    col0 = iota & 0
    col1 = col0 | 1

    @plsc.parallel_loop(0, TBL // LANES, unroll=2)
    def build_body(i):
        r = iota + i * LANES
        pr = r >> 10
        cr = (r >> 9) & 1
        qa = (r & 511) << 1
        v0 = (plsc.load_gather(wp_v, [pr, col0])
              + plsc.load_gather(wc_v, [cr, col0])
              + plsc.load_gather(wq_v, [qa]))
        v1 = (plsc.load_gather(wp_v, [pr, col1])
              + plsc.load_gather(wc_v, [cr, col1])
              + plsc.load_gather(wq_v, [qa | 1]))
        j0 = (iota2 := iota * 2) + i * (2 * LANES)
        plsc.store_scatter(tbl_p, [j0], v0)
        plsc.store_scatter(tbl_p, [j0 | 1], v1)

    out_descs = {}
    for k in range(BLK_PER_W):
        pv, cv, qv, ob, _, sout = bufs[k % NBUF]
        for d in in_descs.pop(k):
            d.wait()
        if k >= NBUF:
            out_descs.pop(k - NBUF).wait()

        @plsc.parallel_loop(0, VECS, unroll=4)
        def vec_body(i):
            l = i >> 3
            o = (i & 7) << 4
            p = pv[l, pl.ds(o, LANES)]
            c = cv[l, pl.ds(o, LANES)]
            q = qv[l, pl.ds(o, LANES)]
            a0 = (p << 11) + (c << 10) + (q << 1)
            v0 = plsc.load_gather(tbl_p, [a0])
            v1 = plsc.load_gather(tbl_p, [a0 | 1])
            ob[l, 0, pl.ds(o, LANES)] = v0
            ob[l, 1, pl.ds(o, LANES)] = v1

        bc = wid * BLK_PER_W + k
        out_descs[k] = pltpu.async_copy(
            ob, out_hbm.at[:, pl.ds(bc * 2, 2), :], sout)
        if k + NBUF < BLK_PER_W:
            in_descs[k + NBUF] = start_in(k + NBUF)

    for k in sorted(out_descs):
        out_descs[k].wait()


@jax.jit
def _run(pT, cT, qT, wp, wc, wq):
    mesh = plsc.VectorSubcoreMesh(core_axis_name="c", subcore_axis_name="s",
                                  num_cores=NC, num_subcores=NS)
    f = pl.kernel(
        _body,
        out_type=jax.ShapeDtypeStruct((L, 2 * N_BLKS, BLK), jnp.float32),
        mesh=mesh,
        compiler_params=pltpu.CompilerParams(needs_layout_passes=False),
        scratch_types=[
            pltpu.VMEM((TBL * 2,), jnp.float32),
            pltpu.VMEM((L, BLK), jnp.int32),
            pltpu.VMEM((L, BLK), jnp.int32),
            pltpu.VMEM((L, BLK), jnp.int32),
            pltpu.VMEM((L, 2, BLK), jnp.float32),
            pltpu.VMEM((L, BLK), jnp.int32),
            pltpu.VMEM((L, BLK), jnp.int32),
            pltpu.VMEM((L, BLK), jnp.int32),
            pltpu.VMEM((L, 2, BLK), jnp.float32),
            pltpu.VMEM((6, 2), jnp.float32),
            pltpu.VMEM((2, 2), jnp.float32),
            pltpu.VMEM((1024,), jnp.float32),
            pltpu.SemaphoreType.DMA,
            pltpu.SemaphoreType.DMA,
            pltpu.SemaphoreType.DMA,
            pltpu.SemaphoreType.DMA,
        ],
    )
    return f(pT, cT, qT, wp, wc, wq)


def kernel(piece_type, color, position, W_piece, W_color, W_pos):
    pT = piece_type.T.astype(jnp.int32)
    cT = color.T.astype(jnp.int32)
    qT = position.T.astype(jnp.int32)
    out_phys = _run(pT, cT, qT, W_piece, W_color,
                    W_pos.reshape(-1))
    out = (out_phys.reshape(L, N_BLKS, 2, BLK)
           .transpose(1, 3, 0, 2)
           .reshape(B, L, 2))
    return out


# speedup vs baseline: 1.0109x; 1.0109x over previous
import jax
import jax.numpy as jnp
from jax.experimental import pallas as pl
from jax.experimental.pallas import tpu as pltpu
from jax.experimental.pallas import tpu_sc as plsc

NC, NS, LANES = 2, 16, 16
NW = NC * NS
B, L = 16384, 64
BLK = 128
N_BLKS = B // BLK
BLK_PER_W = N_BLKS // NW
NBUF = 2
TBL = 6 * 2 * 512
VECS = L * BLK // LANES


def _body(piece_hbm, color_hbm, pos_hbm, wp_hbm, wc_hbm, wq_hbm, out_hbm,
          tbl_p, pv0, cv0, qv0, ob0, pv1, cv1, qv1, ob1,
          wp_v, wc_v, wq_v, sin0, sin1, sout0, sout1):
    wid = jax.lax.axis_index("s") * NC + jax.lax.axis_index("c")
    iota = jax.lax.iota(jnp.int32, LANES)

    bufs = [(pv0, cv0, qv0, ob0, sin0, sout0),
            (pv1, cv1, qv1, ob1, sin1, sout1)]

    def start_in(k):
        pv, cv, qv, _, sin, _ = bufs[k % NBUF]
        bc = (wid * BLK_PER_W + k) * BLK
        sl = pl.ds(bc, BLK)
        return [pltpu.async_copy(piece_hbm.at[:, sl], pv, sin),
                pltpu.async_copy(color_hbm.at[:, sl], cv, sin),
                pltpu.async_copy(pos_hbm.at[:, sl], qv, sin)]

    pltpu.sync_copy(wp_hbm, wp_v)
    pltpu.sync_copy(wc_hbm, wc_v)
    pltpu.sync_copy(wq_hbm, wq_v)

    in_descs = {0: start_in(0), 1: start_in(1)}

    col0 = iota & 0
    col1 = col0 | 1

    @plsc.parallel_loop(0, TBL // LANES, unroll=2)
    def build_body(i):
        r = iota + i * LANES
        pr = r >> 10
        cr = (r >> 9) & 1
        qa = (r & 511) << 1
        v0 = (plsc.load_gather(wp_v, [pr, col0])
              + plsc.load_gather(wc_v, [cr, col0])
              + plsc.load_gather(wq_v, [qa]))
        v1 = (plsc.load_gather(wp_v, [pr, col1])
              + plsc.load_gather(wc_v, [cr, col1])
              + plsc.load_gather(wq_v, [qa | 1]))
        packed = plsc.pack(v0, v1, format=plsc.PackFormat.INTERLEAVED)
        tbl_p[pl.ds(i * LANES, LANES)] = plsc.bitcast(packed, jnp.int32)

    out_descs = {}
    for k in range(BLK_PER_W):
        pv, cv, qv, ob, _, sout = bufs[k % NBUF]
        for d in in_descs.pop(k):
            d.wait()
        if k >= NBUF:
            out_descs.pop(k - NBUF).wait()

        @plsc.parallel_loop(0, VECS, unroll=4)
        def vec_body(i):
            l = i >> 3
            o = (i & 7) << 4
            p = pv[l, pl.ds(o, LANES)]
            c = cv[l, pl.ds(o, LANES)]
            q = qv[l, pl.ds(o, LANES)]
            a = (p << 10) + (c << 9) + q
            w = plsc.load_gather(tbl_p, [a])
            v0, v1 = plsc.unpack(plsc.bitcast(w, jnp.bfloat16),
                                 format=plsc.PackFormat.INTERLEAVED)
            ob[l, 0, pl.ds(o, LANES)] = v0
            ob[l, 1, pl.ds(o, LANES)] = v1

        bc = wid * BLK_PER_W + k
        out_descs[k] = pltpu.async_copy(
            ob, out_hbm.at[:, pl.ds(bc * 2, 2), :], sout)
        if k + NBUF < BLK_PER_W:
            in_descs[k + NBUF] = start_in(k + NBUF)

    for k in sorted(out_descs):
        out_descs[k].wait()


@jax.jit
def _run(pT, cT, qT, wp, wc, wq):
    mesh = plsc.VectorSubcoreMesh(core_axis_name="c", subcore_axis_name="s",
                                  num_cores=NC, num_subcores=NS)
    f = pl.kernel(
        _body,
        out_type=jax.ShapeDtypeStruct((L, 2 * N_BLKS, BLK), jnp.float32),
        mesh=mesh,
        compiler_params=pltpu.CompilerParams(needs_layout_passes=False),
        scratch_types=[
            pltpu.VMEM((TBL,), jnp.int32),
            pltpu.VMEM((L, BLK), jnp.int32),
            pltpu.VMEM((L, BLK), jnp.int32),
            pltpu.VMEM((L, BLK), jnp.int32),
            pltpu.VMEM((L, 2, BLK), jnp.float32),
            pltpu.VMEM((L, BLK), jnp.int32),
            pltpu.VMEM((L, BLK), jnp.int32),
            pltpu.VMEM((L, BLK), jnp.int32),
            pltpu.VMEM((L, 2, BLK), jnp.float32),
            pltpu.VMEM((6, 2), jnp.float32),
            pltpu.VMEM((2, 2), jnp.float32),
            pltpu.VMEM((1024,), jnp.float32),
            pltpu.SemaphoreType.DMA,
            pltpu.SemaphoreType.DMA,
            pltpu.SemaphoreType.DMA,
            pltpu.SemaphoreType.DMA,
        ],
    )
    return f(pT, cT, qT, wp, wc, wq)


def kernel(piece_type, color, position, W_piece, W_color, W_pos):
    pT = piece_type.T.astype(jnp.int32)
    cT = color.T.astype(jnp.int32)
    qT = position.T.astype(jnp.int32)
    out_phys = _run(pT, cT, qT, W_piece, W_color,
                    W_pos.reshape(-1))
    out = (out_phys.reshape(L, N_BLKS, 2, BLK)
           .transpose(1, 3, 0, 2)
           .reshape(B, L, 2))
    return out


# speedup vs baseline: 1.0742x; 1.0627x over previous
import jax
import jax.numpy as jnp
from jax.experimental import pallas as pl
from jax.experimental.pallas import tpu as pltpu
from jax.experimental.pallas import tpu_sc as plsc

NC, NS, LANES = 2, 16, 16
NW = NC * NS
B, L = 16384, 64
BLK = 128
N_BLKS = B // BLK
BLK_PER_W = N_BLKS // NW
NBUF = 2
TBL = 6 * 2 * 512
VECS = L * BLK // LANES


def _body(piece_hbm, color_hbm, pos_hbm, wp_hbm, wc_hbm, wq_hbm, out_hbm,
          tbl_p, pv0, cv0, qv0, ob0, pv1, cv1, qv1, ob1,
          wp_v, wc_v, wq_v, sin0, sin1, sout0, sout1, swt):
    wid = jax.lax.axis_index("s") * NC + jax.lax.axis_index("c")
    iota = jax.lax.iota(jnp.int32, LANES)

    bufs = [(pv0, cv0, qv0, ob0, sin0, sout0),
            (pv1, cv1, qv1, ob1, sin1, sout1)]

    def start_in(k):
        pv, cv, qv, _, sin, _ = bufs[k % NBUF]
        bc = (wid * BLK_PER_W + k) * BLK
        sl = pl.ds(bc, BLK)
        return [pltpu.async_copy(piece_hbm.at[:, sl], pv, sin),
                pltpu.async_copy(color_hbm.at[:, sl], cv, sin),
                pltpu.async_copy(pos_hbm.at[:, sl], qv, sin)]

    w_descs = [pltpu.async_copy(wp_hbm, wp_v, swt),
               pltpu.async_copy(wc_hbm, wc_v, swt),
               pltpu.async_copy(wq_hbm, wq_v, swt)]

    in_descs = {0: start_in(0), 1: start_in(1)}

    for d in w_descs:
        d.wait()

    col0 = iota & 0
    col1 = col0 | 1

    @plsc.parallel_loop(0, TBL // LANES, unroll=2)
    def build_body(i):
        r = iota + i * LANES
        pr = r >> 10
        cr = (r >> 9) & 1
        qa = (r & 511) << 1
        v0 = (plsc.load_gather(wp_v, [pr, col0])
              + plsc.load_gather(wc_v, [cr, col0])
              + plsc.load_gather(wq_v, [qa]))
        v1 = (plsc.load_gather(wp_v, [pr, col1])
              + plsc.load_gather(wc_v, [cr, col1])
              + plsc.load_gather(wq_v, [qa | 1]))
        packed = plsc.pack(v0, v1, format=plsc.PackFormat.INTERLEAVED)
        tbl_p[pl.ds(i * LANES, LANES)] = plsc.bitcast(packed, jnp.int32)

    out_descs = {}
    for k in range(BLK_PER_W):
        pv, cv, qv, ob, _, sout = bufs[k % NBUF]
        for d in in_descs.pop(k):
            d.wait()
        if k >= NBUF:
            out_descs.pop(k - NBUF).wait()

        @plsc.parallel_loop(0, VECS, unroll=4)
        def vec_body(i):
            l = i >> 3
            o = (i & 7) << 4
            p = pv[l, pl.ds(o, LANES)]
            c = cv[l, pl.ds(o, LANES)]
            q = qv[l, pl.ds(o, LANES)]
            a = (p << 10) + (c << 9) + q
            w = plsc.load_gather(tbl_p, [a])
            v0, v1 = plsc.unpack(plsc.bitcast(w, jnp.bfloat16),
                                 format=plsc.PackFormat.INTERLEAVED)
            ob[l, 0, pl.ds(o, LANES)] = v0
            ob[l, 1, pl.ds(o, LANES)] = v1

        bc = wid * BLK_PER_W + k
        out_descs[k] = pltpu.async_copy(
            ob, out_hbm.at[:, pl.ds(bc * 2, 2), :], sout)
        if k + NBUF < BLK_PER_W:
            in_descs[k + NBUF] = start_in(k + NBUF)

    for k in sorted(out_descs):
        out_descs[k].wait()


@jax.jit
def _run(pT, cT, qT, wp, wc, wq):
    mesh = plsc.VectorSubcoreMesh(core_axis_name="c", subcore_axis_name="s",
                                  num_cores=NC, num_subcores=NS)
    f = pl.kernel(
        _body,
        out_type=jax.ShapeDtypeStruct((L, 2 * N_BLKS, BLK), jnp.float32),
        mesh=mesh,
        compiler_params=pltpu.CompilerParams(needs_layout_passes=False),
        scratch_types=[
            pltpu.VMEM((TBL,), jnp.int32),
            pltpu.VMEM((L, BLK), jnp.int32),
            pltpu.VMEM((L, BLK), jnp.int32),
            pltpu.VMEM((L, BLK), jnp.int32),
            pltpu.VMEM((L, 2, BLK), jnp.float32),
            pltpu.VMEM((L, BLK), jnp.int32),
            pltpu.VMEM((L, BLK), jnp.int32),
            pltpu.VMEM((L, BLK), jnp.int32),
            pltpu.VMEM((L, 2, BLK), jnp.float32),
            pltpu.VMEM((6, 2), jnp.float32),
            pltpu.VMEM((2, 2), jnp.float32),
            pltpu.VMEM((1024,), jnp.float32),
            pltpu.SemaphoreType.DMA,
            pltpu.SemaphoreType.DMA,
            pltpu.SemaphoreType.DMA,
            pltpu.SemaphoreType.DMA,
            pltpu.SemaphoreType.DMA,
        ],
    )
    return f(pT, cT, qT, wp, wc, wq)


def kernel(piece_type, color, position, W_piece, W_color, W_pos):
    pT = piece_type.T.astype(jnp.int32)
    cT = color.T.astype(jnp.int32)
    qT = position.T.astype(jnp.int32)
    out_phys = _run(pT, cT, qT, W_piece, W_color,
                    W_pos.reshape(-1))
    out = (out_phys.reshape(L, N_BLKS, 2, BLK)
           .transpose(1, 3, 0, 2)
           .reshape(B, L, 2))
    return out


# speedup vs baseline: 1.0964x; 1.0207x over previous
import jax
import jax.numpy as jnp
from jax.experimental import pallas as pl
from jax.experimental.pallas import tpu as pltpu
from jax.experimental.pallas import tpu_sc as plsc

NC, NS, LANES = 2, 16, 16
NW = NC * NS
B, L = 16384, 64
BLK = 128
N_BLKS = B // BLK
BLK_PER_W = N_BLKS // NW
NBUF = 2
NIN = 3
TBL = 6 * 2 * 512
VECS = L * BLK // LANES


def _body(piece_hbm, color_hbm, pos_hbm, wp_hbm, wc_hbm, wq_hbm, out_hbm,
          tbl_p, pv0, cv0, qv0, pv1, cv1, qv1, pv2, cv2, qv2, ob0, ob1,
          wp_v, wc_v, wq_v, sin0, sin1, sin2, sout0, sout1, swt):
    wid = jax.lax.axis_index("s") * NC + jax.lax.axis_index("c")
    iota = jax.lax.iota(jnp.int32, LANES)

    ibufs = [(pv0, cv0, qv0, sin0), (pv1, cv1, qv1, sin1), (pv2, cv2, qv2, sin2)]
    obufs = [(ob0, sout0), (ob1, sout1)]

    def start_in(k):
        pv, cv, qv, sin = ibufs[k % NIN]
        bc = (wid * BLK_PER_W + k) * BLK
        sl = pl.ds(bc, BLK)
        return [pltpu.async_copy(piece_hbm.at[:, sl], pv, sin),
                pltpu.async_copy(color_hbm.at[:, sl], cv, sin),
                pltpu.async_copy(pos_hbm.at[:, sl], qv, sin)]

    w_descs = [pltpu.async_copy(wp_hbm, wp_v, swt),
               pltpu.async_copy(wc_hbm, wc_v, swt),
               pltpu.async_copy(wq_hbm, wq_v, swt)]

    in_descs = {0: start_in(0), 1: start_in(1), 2: start_in(2)}

    for d in w_descs:
        d.wait()

    col0 = iota & 0
    col1 = col0 | 1

    @plsc.parallel_loop(0, TBL // LANES, unroll=2)
    def build_body(i):
        r = iota + i * LANES
        pr = r >> 10
        cr = (r >> 9) & 1
        qa = (r & 511) << 1
        v0 = (plsc.load_gather(wp_v, [pr, col0])
              + plsc.load_gather(wc_v, [cr, col0])
              + plsc.load_gather(wq_v, [qa]))
        v1 = (plsc.load_gather(wp_v, [pr, col1])
              + plsc.load_gather(wc_v, [cr, col1])
              + plsc.load_gather(wq_v, [qa | 1]))
        packed = plsc.pack(v0, v1, format=plsc.PackFormat.INTERLEAVED)
        tbl_p[pl.ds(i * LANES, LANES)] = plsc.bitcast(packed, jnp.int32)

    out_descs = {}
    for k in range(BLK_PER_W):
        pv, cv, qv, _ = ibufs[k % NIN]
        ob, sout = obufs[k % NBUF]
        for d in in_descs.pop(k):
            d.wait()
        if k >= NBUF:
            out_descs.pop(k - NBUF).wait()

        @plsc.parallel_loop(0, VECS, unroll=4)
        def vec_body(i):
            l = i >> 3
            o = (i & 7) << 4
            p = pv[l, pl.ds(o, LANES)]
            c = cv[l, pl.ds(o, LANES)]
            q = qv[l, pl.ds(o, LANES)]
            a = (p << 10) + (c << 9) + q
            w = plsc.load_gather(tbl_p, [a])
            v0, v1 = plsc.unpack(plsc.bitcast(w, jnp.bfloat16),
                                 format=plsc.PackFormat.INTERLEAVED)
            ob[l, 0, pl.ds(o, LANES)] = v0
            ob[l, 1, pl.ds(o, LANES)] = v1

        bc = wid * BLK_PER_W + k
        out_descs[k] = pltpu.async_copy(
            ob, out_hbm.at[:, pl.ds(bc * 2, 2), :], sout)
        if k + NIN < BLK_PER_W:
            in_descs[k + NIN] = start_in(k + NIN)

    for k in sorted(out_descs):
        out_descs[k].wait()


@jax.jit
def _run(pT, cT, qT, wp, wc, wq):
    mesh = plsc.VectorSubcoreMesh(core_axis_name="c", subcore_axis_name="s",
                                  num_cores=NC, num_subcores=NS)
    f = pl.kernel(
        _body,
        out_type=jax.ShapeDtypeStruct((L, 2 * N_BLKS, BLK), jnp.float32),
        mesh=mesh,
        compiler_params=pltpu.CompilerParams(needs_layout_passes=False),
        scratch_types=[
            pltpu.VMEM((TBL,), jnp.int32),
            pltpu.VMEM((L, BLK), jnp.int32),
            pltpu.VMEM((L, BLK), jnp.int32),
            pltpu.VMEM((L, BLK), jnp.int32),
            pltpu.VMEM((L, BLK), jnp.int32),
            pltpu.VMEM((L, BLK), jnp.int32),
            pltpu.VMEM((L, BLK), jnp.int32),
            pltpu.VMEM((L, BLK), jnp.int32),
            pltpu.VMEM((L, BLK), jnp.int32),
            pltpu.VMEM((L, BLK), jnp.int32),
            pltpu.VMEM((L, 2, BLK), jnp.float32),
            pltpu.VMEM((L, 2, BLK), jnp.float32),
            pltpu.VMEM((6, 2), jnp.float32),
            pltpu.VMEM((2, 2), jnp.float32),
            pltpu.VMEM((1024,), jnp.float32),
            pltpu.SemaphoreType.DMA,
            pltpu.SemaphoreType.DMA,
            pltpu.SemaphoreType.DMA,
            pltpu.SemaphoreType.DMA,
            pltpu.SemaphoreType.DMA,
            pltpu.SemaphoreType.DMA,
        ],
    )
    return f(pT, cT, qT, wp, wc, wq)


def kernel(piece_type, color, position, W_piece, W_color, W_pos):
    pT = piece_type.T.astype(jnp.int32)
    cT = color.T.astype(jnp.int32)
    qT = position.T.astype(jnp.int32)
    out_phys = _run(pT, cT, qT, W_piece, W_color,
                    W_pos.reshape(-1))
    out = (out_phys.reshape(L, N_BLKS, 2, BLK)
           .transpose(1, 3, 0, 2)
           .reshape(B, L, 2))
    return out


# speedup vs baseline: 1.1146x; 1.0166x over previous
import jax
import jax.numpy as jnp
from jax.experimental import pallas as pl
from jax.experimental.pallas import tpu as pltpu
from jax.experimental.pallas import tpu_sc as plsc

NC, NS, LANES = 2, 16, 16
NW = NC * NS
B, L = 16384, 64
BLK = 128
N_BLKS = B // BLK
BLK_PER_W = N_BLKS // NW
NBUF = 2
NIN = 3
TBL = 6 * 2 * 512
VECS = L * BLK // LANES


def _body(piece_hbm, color_hbm, pos_hbm, wp_hbm, wc_hbm, wq_hbm, out_hbm,
          tbl_p, pv0, cv0, qv0, pv1, cv1, qv1, pv2, cv2, qv2, ob0, ob1,
          wp_v, wc_v, wq_v, sin0, sin1, sin2, sout0, sout1, swt):
    wid = jax.lax.axis_index("s") * NC + jax.lax.axis_index("c")
    iota = jax.lax.iota(jnp.int32, LANES)

    ibufs = [(pv0, cv0, qv0, sin0), (pv1, cv1, qv1, sin1), (pv2, cv2, qv2, sin2)]
    obufs = [(ob0, sout0), (ob1, sout1)]

    def start_in(k):
        pv, cv, qv, sin = ibufs[k % NIN]
        bc = (wid * BLK_PER_W + k) * BLK
        sl = pl.ds(bc, BLK)
        return [pltpu.async_copy(piece_hbm.at[:, sl], pv, sin),
                pltpu.async_copy(color_hbm.at[:, sl], cv, sin),
                pltpu.async_copy(pos_hbm.at[:, sl], qv, sin)]

    w_descs = [pltpu.async_copy(wp_hbm, wp_v, swt),
               pltpu.async_copy(wc_hbm, wc_v, swt),
               pltpu.async_copy(wq_hbm, wq_v, swt)]

    in_descs = {0: start_in(0), 1: start_in(1), 2: start_in(2)}

    for d in w_descs:
        d.wait()

    col0 = iota & 0
    col1 = col0 | 1

    @plsc.parallel_loop(0, TBL // LANES, unroll=2)
    def build_body(i):
        r = iota + i * LANES
        pr = r >> 10
        cr = (r >> 9) & 1
        q = r & 511
        qt = q >> 7
        ql = q & 127
        v0 = (plsc.load_gather(wp_v, [col0, pr])
              + plsc.load_gather(wc_v, [cr, col0])
              + plsc.load_gather(wq_v, [qt, col0, ql]))
        v1 = (plsc.load_gather(wp_v, [col1, pr])
              + plsc.load_gather(wc_v, [cr, col1])
              + plsc.load_gather(wq_v, [qt, col1, ql]))
        packed = plsc.pack(v0, v1, format=plsc.PackFormat.INTERLEAVED)
        tbl_p[pl.ds(i * LANES, LANES)] = plsc.bitcast(packed, jnp.int32)

    out_descs = {}
    for k in range(BLK_PER_W):
        pv, cv, qv, _ = ibufs[k % NIN]
        ob, sout = obufs[k % NBUF]
        for d in in_descs.pop(k):
            d.wait()
        if k >= NBUF:
            out_descs.pop(k - NBUF).wait()

        @plsc.parallel_loop(0, VECS, unroll=4)
        def vec_body(i):
            l = i >> 3
            o = (i & 7) << 4
            p = pv[l, pl.ds(o, LANES)]
            c = cv[l, pl.ds(o, LANES)]
            q = qv[l, pl.ds(o, LANES)]
            a = (p << 10) + (c << 9) + q
            w = plsc.load_gather(tbl_p, [a])
            v0, v1 = plsc.unpack(plsc.bitcast(w, jnp.bfloat16),
                                 format=plsc.PackFormat.INTERLEAVED)
            ob[l, 0, pl.ds(o, LANES)] = v0
            ob[l, 1, pl.ds(o, LANES)] = v1

        bc = wid * BLK_PER_W + k
        out_descs[k] = pltpu.async_copy(
            ob, out_hbm.at[:, pl.ds(bc * 2, 2), :], sout)
        if k + NIN < BLK_PER_W:
            in_descs[k + NIN] = start_in(k + NIN)

    for k in sorted(out_descs):
        out_descs[k].wait()


@jax.jit
def _run(pT, cT, qT, wp, wc, wq):
    mesh = plsc.VectorSubcoreMesh(core_axis_name="c", subcore_axis_name="s",
                                  num_cores=NC, num_subcores=NS)
    f = pl.kernel(
        _body,
        out_type=jax.ShapeDtypeStruct((L, 2 * N_BLKS, BLK), jnp.float32),
        mesh=mesh,
        compiler_params=pltpu.CompilerParams(needs_layout_passes=False),
        scratch_types=[
            pltpu.VMEM((TBL,), jnp.int32),
            pltpu.VMEM((L, BLK), jnp.int32),
            pltpu.VMEM((L, BLK), jnp.int32),
            pltpu.VMEM((L, BLK), jnp.int32),
            pltpu.VMEM((L, BLK), jnp.int32),
            pltpu.VMEM((L, BLK), jnp.int32),
            pltpu.VMEM((L, BLK), jnp.int32),
            pltpu.VMEM((L, BLK), jnp.int32),
            pltpu.VMEM((L, BLK), jnp.int32),
            pltpu.VMEM((L, BLK), jnp.int32),
            pltpu.VMEM((L, 2, BLK), jnp.float32),
            pltpu.VMEM((L, 2, BLK), jnp.float32),
            pltpu.VMEM((2, 6), jnp.float32),
            pltpu.VMEM((2, 2), jnp.float32),
            pltpu.VMEM((4, 2, 128), jnp.float32),
            pltpu.SemaphoreType.DMA,
            pltpu.SemaphoreType.DMA,
            pltpu.SemaphoreType.DMA,
            pltpu.SemaphoreType.DMA,
            pltpu.SemaphoreType.DMA,
            pltpu.SemaphoreType.DMA,
        ],
    )
    return f(pT, cT, qT, wp, wc, wq)


def kernel(piece_type, color, position, W_piece, W_color, W_pos):
    pT = piece_type.T.astype(jnp.int32)
    cT = color.T.astype(jnp.int32)
    qT = position.T.astype(jnp.int32)
    wpT = W_piece.T
    wqT = W_pos.reshape(4, 128, 2).transpose(0, 2, 1)
    out_phys = _run(pT, cT, qT, wpT, W_color, wqT)
    out = (out_phys.reshape(L, N_BLKS, 2, BLK)
           .transpose(1, 3, 0, 2)
           .reshape(B, L, 2))
    return out
